# baseline (speedup 1.0000x reference)
import functools

import jax
import jax.numpy as jnp
from jax import lax
from jax.experimental import pallas as pl
from jax.experimental.pallas import tpu as pltpu
from jax.experimental.pallas import tpu_sc as plsc

NC = 2
NS = 16
NW = NC * NS
L = 16

N = 10000
NP = 10240
E = 320000
EP = E
D = 128
BR = 1024
GRID = NP // BR

DEG_CH = 40
AGG_CH = 40
ZR = 64
RPT = NP // NS

_mesh = lambda: plsc.VectorSubcoreMesh(
    core_axis_name="c", subcore_axis_name="s", num_cores=NC, num_subcores=NS)


def _fill_rows(ref, nrows, value, dtype=jnp.float32):
  if dtype == jnp.bfloat16:
    def body(i, _):
      r = pl.multiple_of(2 * i, 2)
      for j in range(D // L):
        ref[pl.ds(r, 2), pl.ds(j * L, L)] = jnp.full((2, L), value, dtype)
      return 0

    lax.fori_loop(0, nrows // 2, body, 0)
    return

  def body(i, _):
    for j in range(D // L):
      ref[i, pl.ds(j * L, L)] = jnp.full((L,), value, dtype)
    return 0

  lax.fori_loop(0, nrows, body, 0)


def _deg_body(src_hbm, dst_hbm, out_hbm, idx_v, ones_v, zbuf_v, deg_sh,
              d0, d1):
  c = lax.axis_index("c")
  s = lax.axis_index("s")

  _fill_rows(ones_v, DEG_CH, 1.0)
  _fill_rows(zbuf_v, ZR, 0.0)
  for j in range(RPT // ZR):
    pltpu.sync_copy(zbuf_v, deg_sh.at[pl.ds(s * RPT + j * ZR, ZR)])

  eper = EP // NS
  half = eper // 2
  for j in range(2):
    @pl.when(c == 0)
    def _():
      pltpu.sync_copy(src_hbm.at[pl.ds(s * eper + j * half, half)],
                      idx_v.at[pl.ds(j * half, half)])

    @pl.when(c == 1)
    def _():
      pltpu.sync_copy(dst_hbm.at[pl.ds(s * eper + j * half, half)],
                      idx_v.at[pl.ds(j * half, half)])

  plsc.subcore_barrier()

  dsem = (d0, d1)

  def dscat(i, b):
    pltpu.async_copy(ones_v, deg_sh.at[idx_v.at[pl.ds(i * DEG_CH, DEG_CH)]],
                     dsem[b], add=True)

  def dwait(b):
    pltpu.make_async_copy(ones_v, deg_sh.at[idx_v.at[pl.ds(0, DEG_CH)]],
                          dsem[b]).wait()

  nit = eper // DEG_CH

  def edge_body(j, _):
    for b in range(2):
      @pl.when(j > 0)
      def _():
        dwait(b)

      dscat(2 * j + b, b)
    return 0

  lax.fori_loop(0, nit // 2, edge_body, 0)
  dwait(0)
  dwait(1)
  plsc.subcore_barrier()

  for j in range(RPT // ZR):
    pltpu.sync_copy(deg_sh.at[pl.ds(s * RPT + j * ZR, ZR)], zbuf_v)
    pltpu.sync_copy(zbuf_v, out_hbm.at[c, pl.ds(s * RPT + j * ZR, ZR)])


def _deg_call(src, dst):
  k = pl.kernel(
      _deg_body,
      out_type=jax.ShapeDtypeStruct((NC, NP, D), jnp.float32),
      mesh=_mesh(),
      scratch_types=[
          pltpu.VMEM((EP // NS,), jnp.int32),
          pltpu.VMEM((DEG_CH, D), jnp.float32),
          pltpu.VMEM((ZR, D), jnp.float32),
          pltpu.VMEM_SHARED((NP, D), jnp.float32),
          pltpu.SemaphoreType.DMA,
          pltpu.SemaphoreType.DMA,
      ],
  )
  return k(src, dst)


def _agg_body(h_hbm, src_hbm, dst_hbm, out_hbm,
              src_v, dst_v, rows_v0, rows_v1, rows_v2, rows_v3, zbuf_v,
              agg_sh, g0, g1, g2, g3, s0, s1, s2, s3):
  c = lax.axis_index("c")
  s = lax.axis_index("s")
  wid = s * NC + c
  rows = (rows_v0, rows_v1, rows_v2, rows_v3)
  gsem = (g0, g1, g2, g3)
  ssem = (s0, s1, s2, s3)

  _fill_rows(zbuf_v, ZR, 0.0)
  for j in range(RPT // ZR):
    pltpu.sync_copy(zbuf_v, agg_sh.at[pl.ds(s * RPT + j * ZR, ZR)])

  eper = EP // NW
  base = wid * eper
  pltpu.sync_copy(src_hbm.at[pl.ds(base, eper)], src_v)
  pltpu.sync_copy(dst_hbm.at[pl.ds(base, eper)], dst_v)
  plsc.subcore_barrier()

  nit = eper // AGG_CH

  def issue(i, b):
    pltpu.async_copy(h_hbm.at[src_v.at[pl.ds(i * AGG_CH, AGG_CH)]],
                     rows[b], gsem[b])

  def wait_g(i, b):
    pltpu.make_async_copy(h_hbm.at[src_v.at[pl.ds(i * AGG_CH, AGG_CH)]],
                          rows[b], gsem[b]).wait()

  def scat(i, b):
    pltpu.async_copy(rows[b], agg_sh.at[dst_v.at[pl.ds(i * AGG_CH, AGG_CH)]],
                     ssem[b], add=True)

  def wait_s(b):
    pltpu.make_async_copy(rows[b], agg_sh.at[dst_v.at[pl.ds(0, AGG_CH)]],
                          ssem[b]).wait()

  issue(0, 0)
  issue(1, 1)

  def quad_body(j, _):
    for b in range(4):
      i = 4 * j + b
      sb = (b + 2) % 4
      wait_g(i, b)
      scat(i, b)
      if b < 2:
        @pl.when(j > 0)
        def _():
          wait_s(sb)
      else:
        wait_s(sb)
      issue(i + 2, sb)
    return 0

  lax.fori_loop(0, (nit - 2) // 4, quad_body, 0)
  for k in range(2):
    i = nit - 2 + k
    wait_g(i, k)
    scat(i, k)
  for b in range(4):
    wait_s(b)
  plsc.subcore_barrier()

  for j in range(RPT // ZR):
    pltpu.sync_copy(agg_sh.at[pl.ds(s * RPT + j * ZR, ZR)], zbuf_v)
    pltpu.sync_copy(zbuf_v, out_hbm.at[c, pl.ds(s * RPT + j * ZR, ZR)])

def _agg_call(h, src, dst):
  k = pl.kernel(
      _agg_body,
      out_type=jax.ShapeDtypeStruct((NC, NP, D), jnp.float32),
      mesh=_mesh(),
      scratch_types=[
          pltpu.VMEM((EP // NW,), jnp.int32),
          pltpu.VMEM((EP // NW,), jnp.int32),
          pltpu.VMEM((AGG_CH, D), jnp.float32),
          pltpu.VMEM((AGG_CH, D), jnp.float32),
          pltpu.VMEM((AGG_CH, D), jnp.float32),
          pltpu.VMEM((AGG_CH, D), jnp.float32),
          pltpu.VMEM((ZR, D), jnp.float32),
          pltpu.VMEM_SHARED((NP, D), jnp.float32),
          pltpu.SemaphoreType.DMA,
          pltpu.SemaphoreType.DMA,
          pltpu.SemaphoreType.DMA,
          pltpu.SemaphoreType.DMA,
          pltpu.SemaphoreType.DMA,
          pltpu.SemaphoreType.DMA,
          pltpu.SemaphoreType.DMA,
          pltpu.SemaphoreType.DMA,
      ],
  )
  return k(h, src, dst)


def _mm1_body(x_ref, deg_ref, w1_ref, h1pre_ref, ns_ref, nd_ref):
  od = deg_ref[0, :, 0]
  idg = deg_ref[1, :, 0]
  ns = jnp.where(od > 0, lax.rsqrt(jnp.maximum(od, 1e-12)), 0.0)
  nd = jnp.where(idg > 0, lax.rsqrt(jnp.maximum(idg, 1e-12)), 0.0)
  ns_ref[...] = ns
  nd_ref[...] = nd
  h1pre_ref[...] = jnp.dot(x_ref[...] * ns[:, None], w1_ref[...],
                           preferred_element_type=jnp.float32)


def _mm1_call(x_pad, deg, W1):
  return pl.pallas_call(
      _mm1_body,
      grid=(GRID,),
      in_specs=[
          pl.BlockSpec((BR, 128), lambda i: (i, 0)),
          pl.BlockSpec((NC, BR, D), lambda i: (0, i, 0)),
          pl.BlockSpec((128, 128), lambda i: (0, 0)),
      ],
      out_specs=[
          pl.BlockSpec((BR, 128), lambda i: (i, 0)),
          pl.BlockSpec((BR,), lambda i: (i,)),
          pl.BlockSpec((BR,), lambda i: (i,)),
      ],
      out_shape=[
          jax.ShapeDtypeStruct((NP, 128), jnp.float32),
          jax.ShapeDtypeStruct((NP,), jnp.float32),
          jax.ShapeDtypeStruct((NP,), jnp.float32),
      ],
  )(x_pad, deg, W1)


def _act_body(aggp_ref, nd_ref, ns_ref, b1_ref, h1n_ref):
  agg = aggp_ref[0] + aggp_ref[1]
  h1 = jnp.maximum(agg * nd_ref[...][:, None] + b1_ref[...][None, :], 0.0)
  h1n_ref[...] = h1 * ns_ref[...][:, None]


def _act_call(agg1p, nd, ns, b1):
  return pl.pallas_call(
      _act_body,
      grid=(GRID,),
      in_specs=[
          pl.BlockSpec((2, BR, 128), lambda i: (0, i, 0)),
          pl.BlockSpec((BR,), lambda i: (i,)),
          pl.BlockSpec((BR,), lambda i: (i,)),
          pl.BlockSpec((128,), lambda i: (0,)),
      ],
      out_specs=pl.BlockSpec((BR, 128), lambda i: (i, 0)),
      out_shape=jax.ShapeDtypeStruct((NP, 128), jnp.float32),
  )(agg1p, nd, ns, b1)


def _mean_body(aggp_ref, nd_ref, b2_ref, w2_ref, out_ref):
  i = pl.program_id(0)

  @pl.when(i == 0)
  def _():
    out_ref[...] = jnp.zeros_like(out_ref)

  agg = aggp_ref[0] + aggp_ref[1]
  t = agg * nd_ref[...][:, None]
  h2 = jnp.dot(t, w2_ref[...], preferred_element_type=jnp.float32)
  h2 = jnp.maximum(h2 + b2_ref[...][None, :], 0.0)
  rows = lax.broadcasted_iota(jnp.int32, (BR, 1), 0) + i * BR
  h2 = jnp.where(rows < N, h2, 0.0)
  out_ref[...] += jnp.sum(h2, axis=0)

  @pl.when(i == GRID - 1)
  def _():
    out_ref[...] = out_ref[...] * (1.0 / N)


def _mean_call(agg2p, nd, b2, W2):
  return pl.pallas_call(
      _mean_body,
      grid=(GRID,),
      in_specs=[
          pl.BlockSpec((2, BR, 128), lambda i: (0, i, 0)),
          pl.BlockSpec((BR,), lambda i: (i,)),
          pl.BlockSpec((32,), lambda i: (0,)),
          pl.BlockSpec((128, 32), lambda i: (0, 0)),
      ],
      out_specs=pl.BlockSpec((32,), lambda i: (0,)),
      out_shape=jax.ShapeDtypeStruct((32,), jnp.float32),
  )(agg2p, nd, b2, W2)


def kernel(x, edge_index, W1, b1, W2, b2):
  src = edge_index[0].astype(jnp.int32)
  dst = edge_index[1].astype(jnp.int32)
  x_pad = jnp.pad(x, ((0, NP - N), (0, 0)))

  deg = _deg_call(src, dst)
  h1pre, ns, nd = _mm1_call(x_pad, deg, W1)
  agg1p = _agg_call(h1pre, src, dst)
  h1n = _act_call(agg1p, nd, ns, b1)
  agg2p = _agg_call(h1n, src, dst)
  return _mean_call(agg2p, nd, b2, W2)


# speedup vs baseline: 1.0021x; 1.0021x over previous
import functools

import jax
import jax.numpy as jnp
from jax import lax
from jax.experimental import pallas as pl
from jax.experimental.pallas import tpu as pltpu
from jax.experimental.pallas import tpu_sc as plsc

NC = 2
NS = 16
NW = NC * NS
L = 16

N = 10000
NP = 10240
E = 320000
EP = E
D = 128
BR = 1024
GRID = NP // BR

DEG_CH = 40
AGG_CH = 40
ZR = 64
RPT = NP // NS

_mesh = lambda: plsc.VectorSubcoreMesh(
    core_axis_name="c", subcore_axis_name="s", num_cores=NC, num_subcores=NS)


def _fill_rows(ref, nrows, value, dtype=jnp.float32):
  if dtype == jnp.bfloat16:
    def body(i, _):
      r = pl.multiple_of(2 * i, 2)
      for j in range(D // L):
        ref[pl.ds(r, 2), pl.ds(j * L, L)] = jnp.full((2, L), value, dtype)
      return 0

    lax.fori_loop(0, nrows // 2, body, 0)
    return

  def body(i, _):
    for j in range(D // L):
      ref[i, pl.ds(j * L, L)] = jnp.full((L,), value, dtype)
    return 0

  lax.fori_loop(0, nrows, body, 0)


def _deg_body(src_hbm, dst_hbm, out_hbm, idx_v, ones_v, zbuf_v, deg_sh,
              d0, d1):
  c = lax.axis_index("c")
  s = lax.axis_index("s")
  dsem = (d0, d1)

  _fill_rows(ones_v, DEG_CH, 1.0)
  _fill_rows(zbuf_v, ZR, 0.0)
  for j in range(RPT // ZR):
    b = j % 2
    if j >= 2:
      pltpu.make_async_copy(zbuf_v, deg_sh.at[pl.ds(0, ZR)], dsem[b]).wait()
    pltpu.async_copy(zbuf_v, deg_sh.at[pl.ds(s * RPT + j * ZR, ZR)], dsem[b])
  for b in range(2):
    pltpu.make_async_copy(zbuf_v, deg_sh.at[pl.ds(0, ZR)], dsem[b]).wait()

  eper = EP // NS
  half = eper // 2
  for j in range(2):
    @pl.when(c == 0)
    def _():
      pltpu.sync_copy(src_hbm.at[pl.ds(s * eper + j * half, half)],
                      idx_v.at[pl.ds(j * half, half)])

    @pl.when(c == 1)
    def _():
      pltpu.sync_copy(dst_hbm.at[pl.ds(s * eper + j * half, half)],
                      idx_v.at[pl.ds(j * half, half)])

  plsc.subcore_barrier()


  def dscat(i, b):
    pltpu.async_copy(ones_v, deg_sh.at[idx_v.at[pl.ds(i * DEG_CH, DEG_CH)]],
                     dsem[b], add=True)

  def dwait(b):
    pltpu.make_async_copy(ones_v, deg_sh.at[idx_v.at[pl.ds(0, DEG_CH)]],
                          dsem[b]).wait()

  nit = eper // DEG_CH

  def edge_body(j, _):
    for b in range(2):
      @pl.when(j > 0)
      def _():
        dwait(b)

      dscat(2 * j + b, b)
    return 0

  lax.fori_loop(0, nit // 2, edge_body, 0)
  dwait(0)
  dwait(1)
  plsc.subcore_barrier()

  for j in range(RPT // ZR):
    pltpu.sync_copy(deg_sh.at[pl.ds(s * RPT + j * ZR, ZR)], zbuf_v)
    pltpu.sync_copy(zbuf_v, out_hbm.at[c, pl.ds(s * RPT + j * ZR, ZR)])


def _deg_call(src, dst):
  k = pl.kernel(
      _deg_body,
      out_type=jax.ShapeDtypeStruct((NC, NP, D), jnp.float32),
      mesh=_mesh(),
      scratch_types=[
          pltpu.VMEM((EP // NS,), jnp.int32),
          pltpu.VMEM((DEG_CH, D), jnp.float32),
          pltpu.VMEM((ZR, D), jnp.float32),
          pltpu.VMEM_SHARED((NP, D), jnp.float32),
          pltpu.SemaphoreType.DMA,
          pltpu.SemaphoreType.DMA,
      ],
  )
  return k(src, dst)


def _agg_body(h_hbm, src_hbm, dst_hbm, out_hbm,
              src_v, dst_v, rows_v0, rows_v1, rows_v2, rows_v3, zbuf_v,
              agg_sh, g0, g1, g2, g3, s0, s1, s2, s3):
  c = lax.axis_index("c")
  s = lax.axis_index("s")
  wid = s * NC + c
  rows = (rows_v0, rows_v1, rows_v2, rows_v3)
  gsem = (g0, g1, g2, g3)
  ssem = (s0, s1, s2, s3)

  _fill_rows(zbuf_v, ZR, 0.0)
  for j in range(RPT // ZR):
    b = j % 2
    if j >= 2:
      pltpu.make_async_copy(zbuf_v, agg_sh.at[pl.ds(0, ZR)], gsem[b]).wait()
    pltpu.async_copy(zbuf_v, agg_sh.at[pl.ds(s * RPT + j * ZR, ZR)], gsem[b])
  for b in range(2):
    pltpu.make_async_copy(zbuf_v, agg_sh.at[pl.ds(0, ZR)], gsem[b]).wait()

  eper = EP // NW
  base = wid * eper
  pltpu.sync_copy(src_hbm.at[pl.ds(base, eper)], src_v)
  pltpu.sync_copy(dst_hbm.at[pl.ds(base, eper)], dst_v)
  plsc.subcore_barrier()

  nit = eper // AGG_CH

  def issue(i, b):
    pltpu.async_copy(h_hbm.at[src_v.at[pl.ds(i * AGG_CH, AGG_CH)]],
                     rows[b], gsem[b])

  def wait_g(i, b):
    pltpu.make_async_copy(h_hbm.at[src_v.at[pl.ds(i * AGG_CH, AGG_CH)]],
                          rows[b], gsem[b]).wait()

  def scat(i, b):
    pltpu.async_copy(rows[b], agg_sh.at[dst_v.at[pl.ds(i * AGG_CH, AGG_CH)]],
                     ssem[b], add=True)

  def wait_s(b):
    pltpu.make_async_copy(rows[b], agg_sh.at[dst_v.at[pl.ds(0, AGG_CH)]],
                          ssem[b]).wait()

  issue(0, 0)
  issue(1, 1)

  def quad_body(j, _):
    for b in range(4):
      i = 4 * j + b
      sb = (b + 2) % 4
      wait_g(i, b)
      scat(i, b)
      if b < 2:
        @pl.when(j > 0)
        def _():
          wait_s(sb)
      else:
        wait_s(sb)
      issue(i + 2, sb)
    return 0

  lax.fori_loop(0, (nit - 2) // 4, quad_body, 0)
  for k in range(2):
    i = nit - 2 + k
    wait_g(i, k)
    scat(i, k)
  for b in range(4):
    wait_s(b)
  plsc.subcore_barrier()

  for j in range(RPT // ZR):
    pltpu.sync_copy(agg_sh.at[pl.ds(s * RPT + j * ZR, ZR)], zbuf_v)
    pltpu.sync_copy(zbuf_v, out_hbm.at[c, pl.ds(s * RPT + j * ZR, ZR)])

def _agg_call(h, src, dst):
  k = pl.kernel(
      _agg_body,
      out_type=jax.ShapeDtypeStruct((NC, NP, D), jnp.float32),
      mesh=_mesh(),
      scratch_types=[
          pltpu.VMEM((EP // NW,), jnp.int32),
          pltpu.VMEM((EP // NW,), jnp.int32),
          pltpu.VMEM((AGG_CH, D), jnp.float32),
          pltpu.VMEM((AGG_CH, D), jnp.float32),
          pltpu.VMEM((AGG_CH, D), jnp.float32),
          pltpu.VMEM((AGG_CH, D), jnp.float32),
          pltpu.VMEM((ZR, D), jnp.float32),
          pltpu.VMEM_SHARED((NP, D), jnp.float32),
          pltpu.SemaphoreType.DMA,
          pltpu.SemaphoreType.DMA,
          pltpu.SemaphoreType.DMA,
          pltpu.SemaphoreType.DMA,
          pltpu.SemaphoreType.DMA,
          pltpu.SemaphoreType.DMA,
          pltpu.SemaphoreType.DMA,
          pltpu.SemaphoreType.DMA,
      ],
  )
  return k(h, src, dst)


def _mm1_body(x_ref, deg_ref, w1_ref, h1pre_ref, ns_ref, nd_ref):
  od = deg_ref[0, :, 0]
  idg = deg_ref[1, :, 0]
  ns = jnp.where(od > 0, lax.rsqrt(jnp.maximum(od, 1e-12)), 0.0)
  nd = jnp.where(idg > 0, lax.rsqrt(jnp.maximum(idg, 1e-12)), 0.0)
  ns_ref[...] = ns
  nd_ref[...] = nd
  h1pre_ref[...] = jnp.dot(x_ref[...] * ns[:, None], w1_ref[...],
                           preferred_element_type=jnp.float32)


def _mm1_call(x_pad, deg, W1):
  return pl.pallas_call(
      _mm1_body,
      grid=(GRID,),
      in_specs=[
          pl.BlockSpec((BR, 128), lambda i: (i, 0)),
          pl.BlockSpec((NC, BR, D), lambda i: (0, i, 0)),
          pl.BlockSpec((128, 128), lambda i: (0, 0)),
      ],
      out_specs=[
          pl.BlockSpec((BR, 128), lambda i: (i, 0)),
          pl.BlockSpec((BR,), lambda i: (i,)),
          pl.BlockSpec((BR,), lambda i: (i,)),
      ],
      out_shape=[
          jax.ShapeDtypeStruct((NP, 128), jnp.float32),
          jax.ShapeDtypeStruct((NP,), jnp.float32),
          jax.ShapeDtypeStruct((NP,), jnp.float32),
      ],
  )(x_pad, deg, W1)


def _act_body(aggp_ref, nd_ref, ns_ref, b1_ref, h1n_ref):
  agg = aggp_ref[0] + aggp_ref[1]
  h1 = jnp.maximum(agg * nd_ref[...][:, None] + b1_ref[...][None, :], 0.0)
  h1n_ref[...] = h1 * ns_ref[...][:, None]


def _act_call(agg1p, nd, ns, b1):
  return pl.pallas_call(
      _act_body,
      grid=(GRID,),
      in_specs=[
          pl.BlockSpec((2, BR, 128), lambda i: (0, i, 0)),
          pl.BlockSpec((BR,), lambda i: (i,)),
          pl.BlockSpec((BR,), lambda i: (i,)),
          pl.BlockSpec((128,), lambda i: (0,)),
      ],
      out_specs=pl.BlockSpec((BR, 128), lambda i: (i, 0)),
      out_shape=jax.ShapeDtypeStruct((NP, 128), jnp.float32),
  )(agg1p, nd, ns, b1)


def _mean_body(aggp_ref, nd_ref, b2_ref, w2_ref, out_ref):
  i = pl.program_id(0)

  @pl.when(i == 0)
  def _():
    out_ref[...] = jnp.zeros_like(out_ref)

  agg = aggp_ref[0] + aggp_ref[1]
  t = agg * nd_ref[...][:, None]
  h2 = jnp.dot(t, w2_ref[...], preferred_element_type=jnp.float32)
  h2 = jnp.maximum(h2 + b2_ref[...][None, :], 0.0)
  rows = lax.broadcasted_iota(jnp.int32, (BR, 1), 0) + i * BR
  h2 = jnp.where(rows < N, h2, 0.0)
  out_ref[...] += jnp.sum(h2, axis=0)

  @pl.when(i == GRID - 1)
  def _():
    out_ref[...] = out_ref[...] * (1.0 / N)


def _mean_call(agg2p, nd, b2, W2):
  return pl.pallas_call(
      _mean_body,
      grid=(GRID,),
      in_specs=[
          pl.BlockSpec((2, BR, 128), lambda i: (0, i, 0)),
          pl.BlockSpec((BR,), lambda i: (i,)),
          pl.BlockSpec((32,), lambda i: (0,)),
          pl.BlockSpec((128, 32), lambda i: (0, 0)),
      ],
      out_specs=pl.BlockSpec((32,), lambda i: (0,)),
      out_shape=jax.ShapeDtypeStruct((32,), jnp.float32),
  )(agg2p, nd, b2, W2)


def kernel(x, edge_index, W1, b1, W2, b2):
  src = edge_index[0].astype(jnp.int32)
  dst = edge_index[1].astype(jnp.int32)
  x_pad = jnp.pad(x, ((0, NP - N), (0, 0)))

  deg = _deg_call(src, dst)
  h1pre, ns, nd = _mm1_call(x_pad, deg, W1)
  agg1p = _agg_call(h1pre, src, dst)
  h1n = _act_call(agg1p, nd, ns, b1)
  agg2p = _agg_call(h1n, src, dst)
  return _mean_call(agg2p, nd, b2, W2)
